# Initial kernel scaffold; baseline (speedup 1.0000x reference)
import jax
import jax.numpy as jnp
from jax.experimental import pallas as pl


def kernel(field0, field1, flip_indices):
    raise NotImplementedError("write your pallas kernel here")


# speedup vs baseline: 3.7337x; 3.7337x over previous
import jax
import jax.numpy as jnp
from jax import lax
from jax.experimental import pallas as pl
from jax.experimental.pallas import tpu as pltpu


def _body(flip_ref, f0_ref, f1_ref, o0_ref, o1_ref):
    del flip_ref
    shape0 = f0_ref.shape
    shape1 = f1_ref.shape
    W = shape0[-1]
    r = lax.broadcasted_iota(jnp.int32, (W, W), 0)
    c = lax.broadcasted_iota(jnp.int32, (W, W), 1)
    P = jnp.where(r + c == W - 1, 1.0, 0.0).astype(jnp.float32)

    x0 = f0_ref[...].reshape(-1, W)
    o0_ref[...] = lax.dot(x0, P, preferred_element_type=jnp.float32).reshape(shape0)

    x1 = f1_ref[...].reshape(-1, W)
    y1 = lax.dot(x1, P, preferred_element_type=jnp.float32).reshape(shape1)
    o1_ref[:, :, 0:1] = -y1[:, :, 0:1]
    o1_ref[:, :, 1:2] = y1[:, :, 1:2]


def kernel(field0, field1, flip_indices):
    B, K, C0, H, W = field0.shape
    C1 = field1.shape[2]

    def in_map(b, k, flip_ref):
        return (b, flip_ref[k], 0, 0, 0)

    def out_map(b, k, flip_ref):
        return (b, k, 0, 0, 0)

    grid_spec = pltpu.PrefetchScalarGridSpec(
        num_scalar_prefetch=1,
        grid=(B, K),
        in_specs=[
            pl.BlockSpec((1, 1, C0, H, W), in_map),
            pl.BlockSpec((1, 1, C1, H, W), in_map),
        ],
        out_specs=[
            pl.BlockSpec((1, 1, C0, H, W), out_map),
            pl.BlockSpec((1, 1, C1, H, W), out_map),
        ],
    )

    out0, out1 = pl.pallas_call(
        _body,
        grid_spec=grid_spec,
        out_shape=[
            jax.ShapeDtypeStruct(field0.shape, field0.dtype),
            jax.ShapeDtypeStruct(field1.shape, field1.dtype),
        ],
    )(flip_indices, field0, field1)
    return (out0, out1)
